# baseline (speedup 1.0000x reference)
import functools

import jax
import jax.numpy as jnp
from jax import lax
from jax.experimental import pallas as pl
from jax.experimental.pallas import tpu as pltpu
from jax.experimental.pallas import tpu_sc as plsc

_NC = 2
_NS = 16


def _seg_sum_sc(f, edge_r, zeros_acc, *, n, e, b):
    chunks = e // b
    rows_per_tile = chunks // _NS
    n_per_tile = n // _NS
    mesh = plsc.VectorSubcoreMesh(core_axis_name="c", subcore_axis_name="s",
                                  num_cores=_NC, num_subcores=_NS)

    out_type = jax.ShapeDtypeStruct((n, 256), jnp.float32)

    phase_len = 40
    n_phases = rows_per_tile // phase_len
    scratch = [
        pltpu.VMEM((phase_len, b), jnp.int32),
        pltpu.VMEM((phase_len, b), jnp.int32),
        pltpu.VMEM((b, 128), jnp.float32),
        pltpu.VMEM((b, 128), jnp.float32),
        pltpu.VMEM_SHARED((n, 128), jnp.float32),
        pltpu.SemaphoreType.DMA,
        pltpu.SemaphoreType.DMA,
        pltpu.SemaphoreType.DMA,
        pltpu.SemaphoreType.DMA,
    ]

    def body(f_hbm, edge_hbm, z_hbm, out, src_v, dst_v,
             rows_0, rows_1, acc, gs_0, gs_1, ss_0, ss_1):
        c = lax.axis_index("c")
        s = lax.axis_index("s")
        r0 = s * n_per_tile
        bufs = (rows_0, rows_1)
        gs = (gs_0, gs_1)
        ss = (ss_0, ss_1)

        pltpu.sync_copy(z_hbm.at[pl.ds(r0, n_per_tile)],
                        acc.at[pl.ds(r0, n_per_tile)])
        plsc.subcore_barrier()

        def run(col0):
            fcol = f_hbm.at[:, pl.ds(col0, 128)]

            def gather(i, q):
                pltpu.async_copy(fcol.at[src_v.at[i]], bufs[q], gs[q])

            def scatter_start(i, q):
                pltpu.async_copy(bufs[q], acc.at[dst_v.at[i]], ss[q],
                                 add=True)

            def scatter_wait(i, q):
                pltpu.make_async_copy(bufs[q], acc.at[dst_v.at[i]],
                                      ss[q]).wait()

            def phase(p, carry):
                row0 = pl.multiple_of(s * rows_per_tile + p * phase_len, 8)
                pltpu.sync_copy(edge_hbm.at[pl.ds(row0, phase_len)], src_v)
                pltpu.sync_copy(edge_hbm.at[pl.ds(chunks + row0, phase_len)],
                                dst_v)

                gather(0, 0)

                def step(k, carry):
                    for j in range(2):
                        i = 2 * k + j
                        qn = (j + 1) % 2

                        @pl.when(i + 1 < phase_len)
                        def _():
                            @pl.when(i >= 1)
                            def _():
                                scatter_wait(i - 1, qn)

                            gather(i + 1, qn)

                        pltpu.make_async_copy(fcol.at[src_v.at[i]], bufs[j],
                                              gs[j]).wait()
                        scatter_start(i, j)
                    return carry

                lax.fori_loop(0, phase_len // 2, step, 0)
                for q in range(2):
                    scatter_wait(phase_len - 2 + q, q)
                return carry

            lax.fori_loop(0, n_phases, phase, 0)

        @pl.when(c == 0)
        def _():
            run(0)

        @pl.when(c == 1)
        def _():
            run(128)

        plsc.subcore_barrier()

        @pl.when(c == 0)
        def _():
            pltpu.sync_copy(acc.at[pl.ds(r0, n_per_tile)],
                            out.at[pl.ds(r0, n_per_tile), pl.ds(0, 128)])

        @pl.when(c == 1)
        def _():
            pltpu.sync_copy(acc.at[pl.ds(r0, n_per_tile)],
                            out.at[pl.ds(r0, n_per_tile), pl.ds(128, 128)])

    fn = pl.kernel(body, out_type=out_type, mesh=mesh, scratch_types=scratch)
    return fn(f, edge_r, zeros_acc)


def _dense_tc(sseg, f, W_l, b_l, W_r, *, n, relu):
    bs = 1000
    grid = (n // bs,)
    b2 = b_l.reshape(1, -1)
    d = W_l.shape[1]

    W_cat = jnp.concatenate([W_l, W_r], axis=1)

    def body(s_ref, f_ref, wc_ref, bl_ref, o_ref):
        mean = s_ref[...]
        nrm = jnp.sqrt(jnp.sum(mean * mean, axis=1, keepdims=True))
        mean = mean / jnp.maximum(nrm, 1e-12)
        xr = f_ref[...]
        xn = jnp.sqrt(jnp.sum(xr * xr, axis=1, keepdims=True))
        xr = xr / jnp.maximum(xn, 1e-12)
        xcat = jnp.concatenate([mean, xr], axis=1)
        out = lax.dot_general(xcat, wc_ref[...], (((1,), (1,)), ((), ())),
                              preferred_element_type=jnp.float32)
        out = out + bl_ref[...]
        if relu:
            out = jnp.maximum(out, 0.0)
        o_ref[...] = out

    blk = pl.BlockSpec((bs, d), lambda i: (i, 0))
    in_specs = [
        blk, blk,
        pl.BlockSpec((d, 2 * d), lambda i: (0, 0)),
        pl.BlockSpec((1, d), lambda i: (0, 0)),
    ]
    return pl.pallas_call(
        body, grid=grid, in_specs=in_specs, out_specs=blk,
        out_shape=jax.ShapeDtypeStruct((n, d), jnp.float32),
    )(sseg, f, W_cat, b2)


def kernel(x, edge_index, W_l1, b_l1, W_r1, W_l2, b_l2, W_r2):
    n, d = x.shape
    e = edge_index.shape[1]
    b = 128
    n_pad = ((n + 8 * _NS - 1) // (8 * _NS)) * (8 * _NS)
    unit = b * _NS * 80
    e_pad = ((e + unit - 1) // unit) * unit
    chunks = e_pad // b

    pad_blk = jnp.concatenate(
        [jnp.zeros((1, e_pad - e), jnp.int32),
         jnp.full((1, e_pad - e), n, jnp.int32)], axis=0)
    edge_r = jnp.concatenate(
        [edge_index.astype(jnp.int32), pad_blk],
        axis=1).reshape(2 * chunks, b)
    zeros_acc = jnp.zeros((n_pad, 128), jnp.float32)

    s1 = _seg_sum_sc(x, edge_r, zeros_acc, n=n_pad, e=e_pad, b=b)
    h1 = _dense_tc(s1, x, W_l1, b_l1, W_r1, n=n, relu=True)
    s2 = _seg_sum_sc(h1, edge_r, zeros_acc, n=n_pad, e=e_pad, b=b)
    out = _dense_tc(s2, h1, W_l2, b_l2, W_r2, n=n, relu=False)
    return out
